# baseline (speedup 1.0000x reference)
import functools
import jax
import jax.numpy as jnp
from jax import lax
from jax.experimental import pallas as pl
from jax.experimental.pallas import tpu as pltpu
from jax.experimental.pallas import tpu_sc as plsc

B, N, H, W = 8, 256, 512, 512
MAX_ITERS = 20
NC, NS, L = 2, 16, 16
NW = NC * NS
PTS = B * N
PPW = PTS // NW
WPB = N // PPW
GRPS = PPW // L
OFFS = [(dy, dx) for dy in (-1, 0, 1) for dx in (-1, 0, 1)]

M = 4
ROUNDS = MAX_ITERS // M
NWIN = 2 * M + 1
CELLS = NWIN * NWIN
NDMA = (CELLS * PPW + 127) // 128
FLAT = NDMA * 128


def _smooth_body(d_ref, o_ref):
    a = d_ref[0, 0]
    zr = jnp.zeros((1, W), jnp.float32)
    rs = a + jnp.concatenate([a[1:], zr], 0) + jnp.concatenate([zr, a[:-1]], 0)
    zc = jnp.zeros((H, 1), jnp.float32)
    cs = rs + jnp.concatenate([rs[:, 1:], zc], 1) + jnp.concatenate([zc, rs[:, :-1]], 1)
    o_ref[0] = cs * jnp.float32(1.0 / 9.0)


_smooth_call = pl.pallas_call(
    _smooth_body,
    out_shape=jax.ShapeDtypeStruct((B, H, W), jnp.float32),
    grid=(B,),
    in_specs=[pl.BlockSpec((1, 1, H, W), lambda b: (b, 0, 0, 0))],
    out_specs=pl.BlockSpec((1, H, W), lambda b: (b, 0, 0)),
)


def _climb_body(d_hbm, pts_hbm, out_hbm,
                pin, ycur, xcur, y0r, x0r, idx_ref, vals_ref, obuf, sem):
    wid = lax.axis_index("s") * NC + lax.axis_index("c")
    base_pt = wid * PPW
    boff = (wid // WPB) * (H * W)
    lane16 = lax.iota(jnp.int32, L)

    pltpu.sync_copy(pts_hbm.at[pl.ds(base_pt * 2, 2 * PPW)], pin)
    for g in range(GRPS):
        sl = pl.ds(g * L, L)
        pi = (lane16 << 1) + (2 * g * L)
        ycur[sl] = plsc.load_gather(pin, [pi])
        xcur[sl] = plsc.load_gather(pin, [pi + 1])

    def round_body(_, carry):
        for g in range(GRPS):
            sl = pl.ds(g * L, L)
            yv = ycur[sl]
            xv = xcur[sl]
            y0r[sl] = yv
            x0r[sl] = xv
            cyu = [((jnp.clip(yv + (u - M), 0, H - 1)) << 9) + boff
                   for u in range(NWIN)]
            cxv = [jnp.clip(xv + (v - M), 0, W - 1) for v in range(NWIN)]
            for u in range(NWIN):
                for v in range(NWIN):
                    c = u * NWIN + v
                    idx_ref[pl.ds(c * PPW + g * L, L)] = cyu[u] + cxv[v]
            idx_ref[pl.ds(CELLS * PPW + g * L, L)] = cyu[NWIN - 1] + cxv[NWIN - 1]
        copies = [pltpu.async_copy(d_hbm.at[idx_ref.at[pl.ds(d * 128, 128)]],
                                   vals_ref.at[pl.ds(d * 128, 128)], sem)
                  for d in range(NDMA)]
        for c in copies:
            c.wait()

        def local_body(_, carry2):
            for g in range(GRPS):
                sl = pl.ds(g * L, L)
                yv = ycur[sl]
                xv = xcur[sl]
                ym = y0r[sl] - M
                xm = x0r[sl] - M
                pv = lane16 + (g * L)
                bv = bdy = bdx = None
                for k, (dy, dx) in enumerate(OFFS):
                    ay = jnp.clip(yv + dy, 0, H - 1)
                    ax = jnp.clip(xv + dx, 0, W - 1)
                    u = ay - ym
                    v = ax - xm
                    cell = ((u << 3) + u) + v
                    val = plsc.load_gather(vals_ref, [(cell << 6) + pv])
                    if k == 0:
                        bv = val
                        bdy = jnp.full((L,), dy, jnp.int32)
                        bdx = jnp.full((L,), dx, jnp.int32)
                    else:
                        m = val > bv
                        bv = jnp.where(m, val, bv)
                        bdy = jnp.where(m, jnp.int32(dy), bdy)
                        bdx = jnp.where(m, jnp.int32(dx), bdx)
                ycur[sl] = jnp.clip(yv + bdy, 0, H - 1)
                xcur[sl] = jnp.clip(xv + bdx, 0, W - 1)
            return carry2

        lax.fori_loop(0, M, local_body, 0)
        return carry

    lax.fori_loop(0, ROUNDS, round_body, 0)

    for g in range(GRPS):
        sl = pl.ds(g * L, L)
        oi = (lane16 << 1) + (2 * g * L)
        plsc.store_scatter(obuf, [oi], ycur[sl].astype(jnp.float32))
        plsc.store_scatter(obuf, [oi + 1], xcur[sl].astype(jnp.float32))
    pltpu.sync_copy(obuf, out_hbm.at[pl.ds(base_pt * 2, 2 * PPW)])


@functools.cache
def _climb_call():
    return pl.kernel(
        _climb_body,
        out_type=jax.ShapeDtypeStruct((PTS * 2,), jnp.float32),
        mesh=plsc.VectorSubcoreMesh(core_axis_name="c", subcore_axis_name="s",
                                    num_cores=NC, num_subcores=NS),
        scratch_types=[
            pltpu.VMEM((2 * PPW,), jnp.int32),
            pltpu.VMEM((PPW,), jnp.int32),
            pltpu.VMEM((PPW,), jnp.int32),
            pltpu.VMEM((PPW,), jnp.int32),
            pltpu.VMEM((PPW,), jnp.int32),
            pltpu.VMEM((FLAT,), jnp.int32),
            pltpu.VMEM((FLAT,), jnp.float32),
            pltpu.VMEM((2 * PPW,), jnp.float32),
            pltpu.SemaphoreType.DMA,
        ],
        compiler_params=pltpu.CompilerParams(needs_layout_passes=False),
    )


def kernel(depth, points):
    d = _smooth_call(depth)
    d_flat = d.reshape(B * H * W)
    pts_flat = points.reshape(PTS * 2).astype(jnp.int32)
    out = _climb_call()(d_flat, pts_flat)
    return out.reshape(B, N, 2)


# speedup vs baseline: 1.1435x; 1.1435x over previous
import jax
import jax.numpy as jnp
from jax import lax
from jax.experimental import pallas as pl
from jax.experimental.pallas import tpu as pltpu
from jax.experimental.pallas import tpu_sc as plsc

B, N, H, W = 8, 256, 512, 512
MAX_ITERS = 20
NC, NS, L = 2, 16, 16
NW = NC * NS
PTS = B * N
PPW = PTS // NW
WPB = N // PPW
NBR = 9
OFFS = [(dy, dx) for dy in (-1, 0, 1) for dx in (-1, 0, 1)]


def _smooth_body(d_ref, o_ref):
    a = d_ref[0, 0]
    zr = jnp.zeros((1, W), jnp.float32)
    rs = a + jnp.concatenate([a[1:], zr], 0) + jnp.concatenate([zr, a[:-1]], 0)
    zc = jnp.zeros((H, 1), jnp.float32)
    cs = rs + jnp.concatenate([rs[:, 1:], zc], 1) + jnp.concatenate([zc, rs[:, :-1]], 1)
    o_ref[0] = cs * jnp.float32(1.0 / 9.0)


_smooth_call = pl.pallas_call(
    _smooth_body,
    out_shape=jax.ShapeDtypeStruct((B, H, W), jnp.float32),
    grid=(B,),
    in_specs=[pl.BlockSpec((1, 1, H, W), lambda b: (b, 0, 0, 0))],
    out_specs=pl.BlockSpec((1, H, W), lambda b: (b, 0, 0)),
)


def _climb_body(d_hbm, ys_hbm, xs_hbm, yo_hbm, xo_hbm,
                ycur, xcur, idx_ref, vals_ref, yf, xf, sem):
    wid = lax.axis_index("s") * NC + lax.axis_index("c")
    base_pt = wid * PPW
    boff = (wid // WPB) * (H * W)

    pltpu.sync_copy(ys_hbm.at[pl.ds(base_pt, PPW)], ycur)
    pltpu.sync_copy(xs_hbm.at[pl.ds(base_pt, PPW)], xcur)

    def body(_, carry):
        for g in range(PPW // L):
            yv = ycur[pl.ds(g * L, L)]
            xv = xcur[pl.ds(g * L, L)]
            for k, (dy, dx) in enumerate(OFFS):
                ny = jnp.clip(yv + dy, 0, H - 1)
                nx = jnp.clip(xv + dx, 0, W - 1)
                idx_ref[k, pl.ds(g * L, L)] = boff + ny * W + nx
        copies = [pltpu.async_copy(d_hbm.at[idx_ref.at[k]], vals_ref.at[k], sem)
                  for k in range(NBR)]
        for c in copies:
            c.wait()
        for g in range(PPW // L):
            yv = ycur[pl.ds(g * L, L)]
            xv = xcur[pl.ds(g * L, L)]
            bv = vals_ref[0, pl.ds(g * L, L)]
            bdy = jnp.full((L,), OFFS[0][0], jnp.int32)
            bdx = jnp.full((L,), OFFS[0][1], jnp.int32)
            for k in range(1, NBR):
                dy, dx = OFFS[k]
                v = vals_ref[k, pl.ds(g * L, L)]
                m = v > bv
                bv = jnp.where(m, v, bv)
                bdy = jnp.where(m, jnp.int32(dy), bdy)
                bdx = jnp.where(m, jnp.int32(dx), bdx)
            ycur[pl.ds(g * L, L)] = jnp.clip(yv + bdy, 0, H - 1)
            xcur[pl.ds(g * L, L)] = jnp.clip(xv + bdx, 0, W - 1)
        return carry

    lax.fori_loop(0, MAX_ITERS, body, 0)

    for g in range(PPW // L):
        yf[pl.ds(g * L, L)] = ycur[pl.ds(g * L, L)].astype(jnp.float32)
        xf[pl.ds(g * L, L)] = xcur[pl.ds(g * L, L)].astype(jnp.float32)
    pltpu.sync_copy(yf, yo_hbm.at[pl.ds(base_pt, PPW)])
    pltpu.sync_copy(xf, xo_hbm.at[pl.ds(base_pt, PPW)])


import functools


@functools.cache
def _climb_call():
    return pl.kernel(
        _climb_body,
        out_type=(jax.ShapeDtypeStruct((PTS,), jnp.float32),
                  jax.ShapeDtypeStruct((PTS,), jnp.float32)),
        mesh=plsc.VectorSubcoreMesh(core_axis_name="c", subcore_axis_name="s",
                                    num_cores=NC, num_subcores=NS),
        scratch_types=[
            pltpu.VMEM((PPW,), jnp.int32),
            pltpu.VMEM((PPW,), jnp.int32),
            pltpu.VMEM((NBR, PPW), jnp.int32),
            pltpu.VMEM((NBR, PPW), jnp.float32),
            pltpu.VMEM((PPW,), jnp.float32),
            pltpu.VMEM((PPW,), jnp.float32),
            pltpu.SemaphoreType.DMA,
        ],
        compiler_params=pltpu.CompilerParams(use_tc_tiling_on_sc=True),
    )


def kernel(depth, points):
    d = _smooth_call(depth)
    d_flat = d.reshape(B * H * W)
    pts = points.reshape(PTS, 2).astype(jnp.int32)
    yf, xf = _climb_call()(d_flat, pts[:, 0], pts[:, 1])
    return jnp.stack([yf, xf], axis=-1).reshape(B, N, 2)


# speedup vs baseline: 1.2053x; 1.0541x over previous
import functools
import jax
import jax.numpy as jnp
from jax import lax
from jax.experimental import pallas as pl
from jax.experimental.pallas import tpu as pltpu
from jax.experimental.pallas import tpu_sc as plsc

B, N, H, W = 8, 256, 512, 512
MAX_ITERS = 20
NC, NS, L = 2, 16, 16
NW = NC * NS
PTS = B * N
PPW = PTS // NW
WPB = N // PPW
GRPS = PPW // L
NBR = 9
OFFS = [(dy, dx) for dy in (-1, 0, 1) for dx in (-1, 0, 1)]
HPTS = PPW // 2
HFLAT = NBR * HPTS
HG = GRPS // 2


def _smooth_body(d_ref, o_ref):
    a = d_ref[0, 0]
    zr = jnp.zeros((1, W), jnp.float32)
    rs = a + jnp.concatenate([a[1:], zr], 0) + jnp.concatenate([zr, a[:-1]], 0)
    zc = jnp.zeros((H, 1), jnp.float32)
    cs = rs + jnp.concatenate([rs[:, 1:], zc], 1) + jnp.concatenate([zc, rs[:, :-1]], 1)
    o_ref[0] = cs * jnp.float32(1.0 / 9.0)


_smooth_call = pl.pallas_call(
    _smooth_body,
    out_shape=jax.ShapeDtypeStruct((B, H, W), jnp.float32),
    grid=(B,),
    in_specs=[pl.BlockSpec((1, 1, H, W), lambda b: (b, 0, 0, 0))],
    out_specs=pl.BlockSpec((1, H, W), lambda b: (b, 0, 0)),
)


def _climb_body(d_hbm, pts_hbm, out_hbm,
                pin, ycur, xcur, idxA, valsA, idxB, valsB, obuf, semA, semB):
    wid = lax.axis_index("s") * NC + lax.axis_index("c")
    base_pt = wid * PPW
    boff = (wid // WPB) * (H * W)
    lane16 = lax.iota(jnp.int32, L)

    pltpu.sync_copy(pts_hbm.at[pl.ds(base_pt * 2, 2 * PPW)], pin)
    for g in range(GRPS):
        sl = pl.ds(g * L, L)
        pi = (lane16 << 1) + (2 * g * L)
        ycur[sl] = plsc.load_gather(pin, [pi])
        xcur[sl] = plsc.load_gather(pin, [pi + 1])

    halves = ((idxA, valsA, semA, 0), (idxB, valsB, semB, 1))

    def compute_idx(idx_ref, h):
        for g in range(HG):
            g_abs = 2 * h + g
            sl = pl.ds(g_abs * L, L)
            yv = ycur[sl]
            xv = xcur[sl]
            cyd = {dy: (jnp.clip(yv + dy, 0, H - 1) << 9) + boff for dy in (-1, 0, 1)}
            cxd = {dx: jnp.clip(xv + dx, 0, W - 1) for dx in (-1, 0, 1)}
            for k, (dy, dx) in enumerate(OFFS):
                idx_ref[pl.ds(k * HPTS + g * L, L)] = cyd[dy] + cxd[dx]

    def fire(idx_ref, vals_ref, sem):
        return [pltpu.async_copy(d_hbm.at[idx_ref.at[pl.ds(o, 96)]],
                                 vals_ref.at[pl.ds(o, 96)], sem)
                for o in (0, 96, 192)]

    def drain(copies):
        for c in copies:
            c.wait()

    def advance(vals_ref, h):
        for g in range(HG):
            g_abs = 2 * h + g
            sl = pl.ds(g_abs * L, L)
            yv = ycur[sl]
            xv = xcur[sl]
            bv = bdy = bdx = None
            for k, (dy, dx) in enumerate(OFFS):
                val = vals_ref[pl.ds(k * HPTS + g * L, L)]
                if k == 0:
                    bv = val
                    bdy = jnp.full((L,), dy, jnp.int32)
                    bdx = jnp.full((L,), dx, jnp.int32)
                else:
                    m = val > bv
                    bv = jnp.where(m, val, bv)
                    bdy = jnp.where(m, jnp.int32(dy), bdy)
                    bdx = jnp.where(m, jnp.int32(dx), bdx)
            ycur[sl] = jnp.clip(yv + bdy, 0, H - 1)
            xcur[sl] = jnp.clip(xv + bdx, 0, W - 1)

    compute_idx(idxA, 0)
    fire(idxA, valsA, semA)
    compute_idx(idxB, 1)
    fire(idxB, valsB, semB)

    def body2(_, carry):
        drain([pltpu.make_async_copy(d_hbm.at[idxA.at[pl.ds(o, 96)]],
                                     valsA.at[pl.ds(o, 96)], semA)
               for o in (0, 96, 192)])
        advance(valsA, 0)
        compute_idx(idxA, 0)
        fire(idxA, valsA, semA)
        drain([pltpu.make_async_copy(d_hbm.at[idxB.at[pl.ds(o, 96)]],
                                     valsB.at[pl.ds(o, 96)], semB)
               for o in (0, 96, 192)])
        advance(valsB, 1)
        compute_idx(idxB, 1)
        fire(idxB, valsB, semB)
        return carry

    lax.fori_loop(0, MAX_ITERS, body2, 0)

    drain([pltpu.make_async_copy(d_hbm.at[idxA.at[pl.ds(o, 96)]],
                                 valsA.at[pl.ds(o, 96)], semA)
           for o in (0, 96, 192)])
    drain([pltpu.make_async_copy(d_hbm.at[idxB.at[pl.ds(o, 96)]],
                                 valsB.at[pl.ds(o, 96)], semB)
           for o in (0, 96, 192)])

    for g in range(GRPS):
        sl = pl.ds(g * L, L)
        oi = (lane16 << 1) + (2 * g * L)
        plsc.store_scatter(obuf, [oi], ycur[sl].astype(jnp.float32))
        plsc.store_scatter(obuf, [oi + 1], xcur[sl].astype(jnp.float32))
    pltpu.sync_copy(obuf, out_hbm.at[pl.ds(base_pt * 2, 2 * PPW)])


@functools.cache
def _climb_call():
    return pl.kernel(
        _climb_body,
        out_type=jax.ShapeDtypeStruct((PTS * 2,), jnp.float32),
        mesh=plsc.VectorSubcoreMesh(core_axis_name="c", subcore_axis_name="s",
                                    num_cores=NC, num_subcores=NS),
        scratch_types=[
            pltpu.VMEM((2 * PPW,), jnp.int32),
            pltpu.VMEM((PPW,), jnp.int32),
            pltpu.VMEM((PPW,), jnp.int32),
            pltpu.VMEM((HFLAT,), jnp.int32),
            pltpu.VMEM((HFLAT,), jnp.float32),
            pltpu.VMEM((HFLAT,), jnp.int32),
            pltpu.VMEM((HFLAT,), jnp.float32),
            pltpu.VMEM((2 * PPW,), jnp.float32),
            pltpu.SemaphoreType.DMA,
            pltpu.SemaphoreType.DMA,
        ],
        compiler_params=pltpu.CompilerParams(needs_layout_passes=False),
    )


def kernel(depth, points):
    d = _smooth_call(depth)
    d_flat = d.reshape(B * H * W)
    pts_flat = points.reshape(PTS * 2).astype(jnp.int32)
    out = _climb_call()(d_flat, pts_flat)
    return out.reshape(B, N, 2)


# speedup vs baseline: 1.4854x; 1.2324x over previous
import functools
import jax
import jax.numpy as jnp
from jax import lax
from jax.experimental import pallas as pl
from jax.experimental.pallas import tpu as pltpu
from jax.experimental.pallas import tpu_sc as plsc

B, N, H, W = 8, 256, 512, 512
MAX_ITERS = 20
NC, NS, L = 2, 16, 16
NW = NC * NS
PTS = B * N
PPW = PTS // NW
WPB = N // PPW
GRPS = PPW // L
NBR = 9
OFFS = [(dy, dx) for dy in (-1, 0, 1) for dx in (-1, 0, 1)]
HPTS = PPW // 2
HFLAT = NBR * HPTS
HG = GRPS // 2


def _smooth_body(d_ref, o_ref):
    a = d_ref[0, 0]
    zr = jnp.zeros((1, W), jnp.float32)
    rs = a + jnp.concatenate([a[1:], zr], 0) + jnp.concatenate([zr, a[:-1]], 0)
    zc = jnp.zeros((H, 1), jnp.float32)
    cs = rs + jnp.concatenate([rs[:, 1:], zc], 1) + jnp.concatenate([zc, rs[:, :-1]], 1)
    o_ref[0] = cs * jnp.float32(1.0 / 9.0)


_smooth_call = pl.pallas_call(
    _smooth_body,
    out_shape=jax.ShapeDtypeStruct((B, H, W), jnp.float32),
    grid=(B,),
    in_specs=[pl.BlockSpec((1, 1, H, W), lambda b: (b, 0, 0, 0))],
    out_specs=pl.BlockSpec((1, H, W), lambda b: (b, 0, 0)),
)


def _climb_body(d_hbm, pts_hbm, out_hbm,
                pin, ycur, xcur, idxA, valsA, idxB, valsB, obuf, shared,
                semA, semB):
    cid = lax.axis_index("c")
    sid = lax.axis_index("s")
    wid = cid * NS + sid
    base_pt = wid * PPW
    boff = (sid // WPB) * (H * W)

    SEG = WPB * H * W // NS
    pltpu.sync_copy(d_hbm.at[pl.ds(cid * (WPB * H * W) + sid * SEG, SEG)],
                    shared.at[pl.ds(sid * SEG, SEG)])
    plsc.subcore_barrier()

    lane16 = lax.iota(jnp.int32, L)

    pltpu.sync_copy(pts_hbm.at[pl.ds(base_pt * 2, 2 * PPW)], pin)
    for g in range(GRPS):
        sl = pl.ds(g * L, L)
        pi = (lane16 << 1) + (2 * g * L)
        ycur[sl] = plsc.load_gather(pin, [pi])
        xcur[sl] = plsc.load_gather(pin, [pi + 1])

    halves = ((idxA, valsA, semA, 0), (idxB, valsB, semB, 1))

    def compute_idx(idx_ref, h):
        for g in range(HG):
            g_abs = 2 * h + g
            sl = pl.ds(g_abs * L, L)
            yv = ycur[sl]
            xv = xcur[sl]
            cyd = {dy: (jnp.clip(yv + dy, 0, H - 1) << 9) + boff for dy in (-1, 0, 1)}
            cxd = {dx: jnp.clip(xv + dx, 0, W - 1) for dx in (-1, 0, 1)}
            for k, (dy, dx) in enumerate(OFFS):
                idx_ref[pl.ds(k * HPTS + g * L, L)] = cyd[dy] + cxd[dx]

    def fire(idx_ref, vals_ref, sem):
        return [pltpu.async_copy(shared.at[idx_ref.at[pl.ds(o, 96)]],
                                 vals_ref.at[pl.ds(o, 96)], sem)
                for o in (0, 96, 192)]

    def drain(copies):
        for c in copies:
            c.wait()

    def advance(vals_ref, h):
        for g in range(HG):
            g_abs = 2 * h + g
            sl = pl.ds(g_abs * L, L)
            yv = ycur[sl]
            xv = xcur[sl]
            bv = bdy = bdx = None
            for k, (dy, dx) in enumerate(OFFS):
                val = vals_ref[pl.ds(k * HPTS + g * L, L)]
                if k == 0:
                    bv = val
                    bdy = jnp.full((L,), dy, jnp.int32)
                    bdx = jnp.full((L,), dx, jnp.int32)
                else:
                    m = val > bv
                    bv = jnp.where(m, val, bv)
                    bdy = jnp.where(m, jnp.int32(dy), bdy)
                    bdx = jnp.where(m, jnp.int32(dx), bdx)
            ycur[sl] = jnp.clip(yv + bdy, 0, H - 1)
            xcur[sl] = jnp.clip(xv + bdx, 0, W - 1)

    compute_idx(idxA, 0)
    fire(idxA, valsA, semA)
    compute_idx(idxB, 1)
    fire(idxB, valsB, semB)

    def body2(_, carry):
        drain([pltpu.make_async_copy(shared.at[idxA.at[pl.ds(o, 96)]],
                                     valsA.at[pl.ds(o, 96)], semA)
               for o in (0, 96, 192)])
        advance(valsA, 0)
        compute_idx(idxA, 0)
        fire(idxA, valsA, semA)
        drain([pltpu.make_async_copy(shared.at[idxB.at[pl.ds(o, 96)]],
                                     valsB.at[pl.ds(o, 96)], semB)
               for o in (0, 96, 192)])
        advance(valsB, 1)
        compute_idx(idxB, 1)
        fire(idxB, valsB, semB)
        return carry

    lax.fori_loop(0, MAX_ITERS, body2, 0)

    drain([pltpu.make_async_copy(shared.at[idxA.at[pl.ds(o, 96)]],
                                 valsA.at[pl.ds(o, 96)], semA)
           for o in (0, 96, 192)])
    drain([pltpu.make_async_copy(shared.at[idxB.at[pl.ds(o, 96)]],
                                 valsB.at[pl.ds(o, 96)], semB)
           for o in (0, 96, 192)])

    for g in range(GRPS):
        sl = pl.ds(g * L, L)
        oi = (lane16 << 1) + (2 * g * L)
        plsc.store_scatter(obuf, [oi], ycur[sl].astype(jnp.float32))
        plsc.store_scatter(obuf, [oi + 1], xcur[sl].astype(jnp.float32))
    pltpu.sync_copy(obuf, out_hbm.at[pl.ds(base_pt * 2, 2 * PPW)])


@functools.cache
def _climb_call():
    return pl.kernel(
        _climb_body,
        out_type=jax.ShapeDtypeStruct((PTS * 2,), jnp.float32),
        mesh=plsc.VectorSubcoreMesh(core_axis_name="c", subcore_axis_name="s",
                                    num_cores=NC, num_subcores=NS),
        scratch_types=[
            pltpu.VMEM((2 * PPW,), jnp.int32),
            pltpu.VMEM((PPW,), jnp.int32),
            pltpu.VMEM((PPW,), jnp.int32),
            pltpu.VMEM((HFLAT,), jnp.int32),
            pltpu.VMEM((HFLAT,), jnp.float32),
            pltpu.VMEM((HFLAT,), jnp.int32),
            pltpu.VMEM((HFLAT,), jnp.float32),
            pltpu.VMEM((2 * PPW,), jnp.float32),
            pltpu.VMEM_SHARED((WPB * H * W,), jnp.float32),
            pltpu.SemaphoreType.DMA,
            pltpu.SemaphoreType.DMA,
        ],
        compiler_params=pltpu.CompilerParams(needs_layout_passes=False),
    )


def kernel(depth, points):
    d = _smooth_call(depth)
    d_flat = d.reshape(B * H * W)
    pts_flat = points.reshape(PTS * 2).astype(jnp.int32)
    out = _climb_call()(d_flat, pts_flat)
    return out.reshape(B, N, 2)


# speedup vs baseline: 1.8109x; 1.2191x over previous
import functools
import jax
import jax.numpy as jnp
from jax import lax
from jax.experimental import pallas as pl
from jax.experimental.pallas import tpu as pltpu
from jax.experimental.pallas import tpu_sc as plsc

B, N, H, W = 8, 256, 512, 512
MAX_ITERS = 20
NC, NS, L = 2, 16, 16
NW = NC * NS
PTS = B * N
PPW = PTS // NW
WPB = N // PPW
GRPS = PPW // L
NBR = 9
OFFS = [(dy, dx) for dy in (-1, 0, 1) for dx in (-1, 0, 1)]
HPTS = PPW // 2
HFLAT = NBR * HPTS
HG = GRPS // 2


def _smooth_body(d_ref, o_ref):
    a = d_ref[0, 0]
    zr = jnp.zeros((1, W), jnp.float32)
    rs = a + jnp.concatenate([a[1:], zr], 0) + jnp.concatenate([zr, a[:-1]], 0)
    zc = jnp.zeros((H, 1), jnp.float32)
    cs = rs + jnp.concatenate([rs[:, 1:], zc], 1) + jnp.concatenate([zc, rs[:, :-1]], 1)
    o_ref[0] = (cs * jnp.float32(1.0 / 9.0)).reshape(H * 4, 128)


_smooth_call = pl.pallas_call(
    _smooth_body,
    out_shape=jax.ShapeDtypeStruct((B, H * 4, 128), jnp.float32),
    grid=(B,),
    in_specs=[pl.BlockSpec((1, 1, H, W), lambda b: (b, 0, 0, 0))],
    out_specs=pl.BlockSpec((1, H * 4, 128), lambda b: (b, 0, 0)),
)


def _climb_body(d_hbm, pts_hbm, out_hbm,
                pin, ycur, xcur, idxA, valsA, idxB, valsB, obuf, shared,
                semA, semB):
    cid = lax.axis_index("c")
    sid = lax.axis_index("s")
    wid = cid * NS + sid
    base_pt = wid * PPW
    boff = (sid // WPB) * (H * W)

    SEG = WPB * H * W // NS
    pltpu.sync_copy(d_hbm.at[pl.ds(cid * (WPB * H * W) + sid * SEG, SEG)],
                    shared.at[pl.ds(sid * SEG, SEG)])
    plsc.subcore_barrier()

    lane16 = lax.iota(jnp.int32, L)

    pltpu.sync_copy(pts_hbm.at[pl.ds(base_pt * 2, 2 * PPW)], pin)
    for g in range(GRPS):
        sl = pl.ds(g * L, L)
        pi = (lane16 << 1) + (2 * g * L)
        ycur[sl] = plsc.load_gather(pin, [pi])
        xcur[sl] = plsc.load_gather(pin, [pi + 1])

    halves = ((idxA, valsA, semA, 0), (idxB, valsB, semB, 1))

    def compute_idx(idx_ref, h):
        for g in range(HG):
            g_abs = 2 * h + g
            sl = pl.ds(g_abs * L, L)
            yv = ycur[sl]
            xv = xcur[sl]
            cyd = {dy: (jnp.clip(yv + dy, 0, H - 1) << 9) + boff for dy in (-1, 0, 1)}
            cxd = {dx: jnp.clip(xv + dx, 0, W - 1) for dx in (-1, 0, 1)}
            for k, (dy, dx) in enumerate(OFFS):
                idx_ref[pl.ds(k * HPTS + g * L, L)] = cyd[dy] + cxd[dx]

    def fire(idx_ref, vals_ref, sem):
        return [pltpu.async_copy(shared.at[idx_ref.at[pl.ds(o, 96)]],
                                 vals_ref.at[pl.ds(o, 96)], sem)
                for o in (0, 96, 192)]

    def drain(copies):
        for c in copies:
            c.wait()

    def advance(vals_ref, h):
        for g in range(HG):
            g_abs = 2 * h + g
            sl = pl.ds(g_abs * L, L)
            yv = ycur[sl]
            xv = xcur[sl]
            bv = bdy = bdx = None
            for k, (dy, dx) in enumerate(OFFS):
                val = vals_ref[pl.ds(k * HPTS + g * L, L)]
                if k == 0:
                    bv = val
                    bdy = jnp.full((L,), dy, jnp.int32)
                    bdx = jnp.full((L,), dx, jnp.int32)
                else:
                    m = val > bv
                    bv = jnp.where(m, val, bv)
                    bdy = jnp.where(m, jnp.int32(dy), bdy)
                    bdx = jnp.where(m, jnp.int32(dx), bdx)
            ycur[sl] = jnp.clip(yv + bdy, 0, H - 1)
            xcur[sl] = jnp.clip(xv + bdx, 0, W - 1)

    compute_idx(idxA, 0)
    fire(idxA, valsA, semA)
    compute_idx(idxB, 1)
    fire(idxB, valsB, semB)

    def body2(_, carry):
        drain([pltpu.make_async_copy(shared.at[idxA.at[pl.ds(o, 96)]],
                                     valsA.at[pl.ds(o, 96)], semA)
               for o in (0, 96, 192)])
        advance(valsA, 0)
        compute_idx(idxA, 0)
        fire(idxA, valsA, semA)
        drain([pltpu.make_async_copy(shared.at[idxB.at[pl.ds(o, 96)]],
                                     valsB.at[pl.ds(o, 96)], semB)
               for o in (0, 96, 192)])
        advance(valsB, 1)
        compute_idx(idxB, 1)
        fire(idxB, valsB, semB)
        return carry

    lax.fori_loop(0, MAX_ITERS, body2, 0)

    drain([pltpu.make_async_copy(shared.at[idxA.at[pl.ds(o, 96)]],
                                 valsA.at[pl.ds(o, 96)], semA)
           for o in (0, 96, 192)])
    drain([pltpu.make_async_copy(shared.at[idxB.at[pl.ds(o, 96)]],
                                 valsB.at[pl.ds(o, 96)], semB)
           for o in (0, 96, 192)])

    for g in range(GRPS):
        sl = pl.ds(g * L, L)
        oi = (lane16 << 1) + (2 * g * L)
        plsc.store_scatter(obuf, [oi], ycur[sl].astype(jnp.float32))
        plsc.store_scatter(obuf, [oi + 1], xcur[sl].astype(jnp.float32))
    pltpu.sync_copy(obuf, out_hbm.at[pl.ds(base_pt * 2, 2 * PPW)])


@functools.cache
def _climb_call():
    return pl.kernel(
        _climb_body,
        out_type=jax.ShapeDtypeStruct((PTS * 2,), jnp.float32),
        mesh=plsc.VectorSubcoreMesh(core_axis_name="c", subcore_axis_name="s",
                                    num_cores=NC, num_subcores=NS),
        scratch_types=[
            pltpu.VMEM((2 * PPW,), jnp.int32),
            pltpu.VMEM((PPW,), jnp.int32),
            pltpu.VMEM((PPW,), jnp.int32),
            pltpu.VMEM((HFLAT,), jnp.int32),
            pltpu.VMEM((HFLAT,), jnp.float32),
            pltpu.VMEM((HFLAT,), jnp.int32),
            pltpu.VMEM((HFLAT,), jnp.float32),
            pltpu.VMEM((2 * PPW,), jnp.float32),
            pltpu.VMEM_SHARED((WPB * H * W,), jnp.float32),
            pltpu.SemaphoreType.DMA,
            pltpu.SemaphoreType.DMA,
        ],
        compiler_params=pltpu.CompilerParams(needs_layout_passes=False),
    )


def kernel(depth, points):
    d = _smooth_call(depth)
    d_flat = d.reshape(B * H * W)
    pts_flat = points.reshape(PTS * 2).astype(jnp.int32)
    out = _climb_call()(d_flat, pts_flat)
    return out.reshape(B, N, 2)
